# baseline (speedup 1.0000x reference)
import functools
import math

import jax
import jax.numpy as jnp
from jax import lax
from jax.experimental import pallas as pl
from jax.experimental.pallas import tpu as pltpu
from jax.experimental.pallas import tpu_sc as plsc

_NB = (24, 12, 16)
_PI = math.pi

_B, _T, _C = 16384, 512, 3
_TOK = _B * _T
_F = _TOK * 3
_MW = _TOK // 2

_NCORES, _NSUB, _L = 2, 16, 16
_NWORK = _NCORES * _NSUB
_TPW = _TOK // _NWORK

_CHUNK = 4096
_FC = _CHUNK * 3
_MC = _CHUNK // 2
_NCH = _TPW // _CHUNK
_BPC = _CHUNK // 32


def _vgather(x, idx):
    dn = lax.GatherDimensionNumbers(
        offset_dims=(), collapsed_slice_dims=(0,), start_index_map=(0,))
    return lax.gather(x, idx[:, None], dn, slice_sizes=(1,),
                      mode=lax.GatherScatterMode.PROMISE_IN_BOUNDS)


def _phase_consts():
    l = lax.iota(jnp.int32, _L)
    out = []
    for j in range(3):
        p = 16 * j + l
        c = p % 3
        tok = ((p - c).astype(jnp.float32) * (1.0 / 3.0)).astype(jnp.int32)
        is0 = c == 0
        is1 = c == 1
        inv = jnp.where(is0, _NB[0] / (2 * _PI),
                        jnp.where(is1, _NB[1] / (2 * _PI), _NB[2] / (2 * _PI)))
        halfn = jnp.where(is0, _NB[0] * 0.5,
                          jnp.where(is1, _NB[1] * 0.5, _NB[2] * 0.5))
        width = jnp.where(is0, 2 * _PI / _NB[0],
                          jnp.where(is1, 2 * _PI / _NB[1], 2 * _PI / _NB[2]))
        nm1 = jnp.where(is0, _NB[0] - 1, jnp.where(is1, _NB[1] - 1, _NB[2] - 1))
        nval = jnp.where(is0, _NB[0], _NB[1])
        mbyte = 2 * tok + c
        sel = mbyte & 3
        bmask = jnp.where(sel == 0, 1,
                          jnp.where(sel == 1, 0x100,
                                    jnp.where(sel == 2, 0x10000, 0x1000000)))
        bmask = jnp.where(c < 2, bmask, 0)
        widx = jnp.where(c < 2, lax.shift_right_logical(mbyte, 2), 0)
        out.append((inv.astype(jnp.float32), halfn.astype(jnp.float32),
                    width.astype(jnp.float32), nm1, nval, bmask, widx))
    return out


def _sc_body(ang_hbm, msk_hbm, q_hbm, i_hbm, ang_v, msk_v, q_v, i_v):
    wid = lax.axis_index("s") * _NCORES + lax.axis_index("c")
    fbase = wid * (_TPW * 3)
    mbase = wid * (_TPW // 2)

    ph = _phase_consts()

    def block(b, carry):
        boff = b * 96
        mw = msk_v[pl.ds(b * 16, 16)]
        for v in range(6):
            j, g = v % 3, v // 3
            inv, halfn, width, nm1, nval, bmask, widx = ph[j]
            a = ang_v[pl.ds(boff + 16 * v, 16)]
            mg = _vgather(mw, widx + 8 * g)
            m = (mg & bmask) != 0
            t = a * inv + halfn
            i = t.astype(jnp.int32)
            i = jnp.minimum(i, nm1)
            q = (i.astype(jnp.float32) + 0.5) * width - _PI
            q_v[pl.ds(boff + 16 * v, 16)] = jnp.where(m, 0.0, q)
            i_v[pl.ds(boff + 16 * v, 16)] = jnp.where(m, nval, i)
        return carry

    def chunk(k, carry):
        foff = fbase + k * _FC
        moff = mbase + k * _MC
        pltpu.sync_copy(ang_hbm.at[pl.ds(foff, _FC)], ang_v)
        pltpu.sync_copy(msk_hbm.at[pl.ds(moff, _MC)], msk_v)
        lax.fori_loop(0, _BPC, block, 0)
        pltpu.sync_copy(q_v, q_hbm.at[pl.ds(foff, _FC)])
        pltpu.sync_copy(i_v, i_hbm.at[pl.ds(foff, _FC)])
        return carry

    lax.fori_loop(0, _NCH, chunk, 0)


_mesh = plsc.VectorSubcoreMesh(core_axis_name="c", subcore_axis_name="s",
                               num_cores=_NCORES, num_subcores=_NSUB)

_sc_call = functools.partial(
    pl.kernel,
    out_type=(jax.ShapeDtypeStruct((_F,), jnp.float32),
              jax.ShapeDtypeStruct((_F,), jnp.int32)),
    mesh=_mesh,
    scratch_types=[
        pltpu.VMEM((_FC,), jnp.float32),
        pltpu.VMEM((_MC,), jnp.int32),
        pltpu.VMEM((_FC,), jnp.float32),
        pltpu.VMEM((_FC,), jnp.int32),
    ],
)(_sc_body)


def kernel(angles, null_mask):
    a_flat = angles.reshape(_F)
    m_words = jax.lax.bitcast_convert_type(
        null_mask.view(jnp.uint8).reshape(_MW, 4), jnp.int32)
    q_flat, i_flat = _sc_call(a_flat, m_words)
    return (q_flat.reshape(_B, _T, _C), i_flat.reshape(_B, _T, _C))


# speedup vs baseline: 63.0212x; 63.0212x over previous
import functools
import math

import jax
import jax.numpy as jnp
import numpy as np
from jax import lax
from jax.experimental import pallas as pl
from jax.experimental.pallas import tpu as pltpu
from jax.experimental.pallas import tpu_sc as plsc

_NB = (24, 12, 16)
_PI = math.pi

_B, _T, _C = 16384, 512, 3
_PLANE = _B * _T
_F = _PLANE * _C
_M = _PLANE * 2

_NCORES, _NSUB = 2, 16
_NWORK = _NCORES * _NSUB
_SLAB = _PLANE // _NWORK

_CHUNK = 16384
_NCH = _SLAB // _CHUNK
_UNROLL = 8
_NVEC = _CHUNK // (16 * _UNROLL)


def _sc_body(ang_hbm, msk_hbm, q_hbm, i_hbm, ang_v, msk_v, q_v, i_v):
    wid = lax.axis_index("s") * _NCORES + lax.axis_index("c")
    sbase = wid * _SLAB

    for ci in range(3):
        n = _NB[ci]
        inv = float(np.float32(n / (2 * _PI)))
        halfn = float(np.float32(n * 0.5))
        width = float(np.float32(2 * _PI / n))
        pbase = ci * _PLANE + sbase

        def chunk(k, carry, ci=ci, n=n, inv=inv, halfn=halfn, width=width,
                  pbase=pbase):
            off = pbase + k * _CHUNK
            pltpu.sync_copy(ang_hbm.at[pl.ds(off, _CHUNK)], ang_v)
            if ci < 2:
                pltpu.sync_copy(msk_hbm.at[pl.ds(off, _CHUNK)], msk_v)

            def vec(b, c2, ci=ci, n=n, inv=inv, halfn=halfn, width=width):
                for u in range(_UNROLL):
                    o = b * (16 * _UNROLL) + u * 16
                    a = ang_v[pl.ds(o, 16)]
                    t = a * inv + halfn
                    i = jnp.minimum(t.astype(jnp.int32), n - 1)
                    q = (i.astype(jnp.float32) + 0.5) * width - _PI
                    if ci < 2:
                        m = msk_v[pl.ds(o, 16)] != 0
                        q = jnp.where(m, 0.0, q)
                        i = jnp.where(m, n, i)
                    q_v[pl.ds(o, 16)] = q
                    i_v[pl.ds(o, 16)] = i
                return c2

            lax.fori_loop(0, _NVEC, vec, 0)
            pltpu.sync_copy(q_v, q_hbm.at[pl.ds(off, _CHUNK)])
            pltpu.sync_copy(i_v, i_hbm.at[pl.ds(off, _CHUNK)])
            return carry

        lax.fori_loop(0, _NCH, chunk, 0)


_mesh = plsc.VectorSubcoreMesh(core_axis_name="c", subcore_axis_name="s",
                               num_cores=_NCORES, num_subcores=_NSUB)

_sc_call = functools.partial(
    pl.kernel,
    out_type=(jax.ShapeDtypeStruct((_F,), jnp.float32),
              jax.ShapeDtypeStruct((_F,), jnp.int32)),
    mesh=_mesh,
    scratch_types=[
        pltpu.VMEM((_CHUNK,), jnp.float32),
        pltpu.VMEM((_CHUNK,), jnp.int32),
        pltpu.VMEM((_CHUNK,), jnp.float32),
        pltpu.VMEM((_CHUNK,), jnp.int32),
    ],
)(_sc_body)


def kernel(angles, null_mask):
    a = jnp.transpose(angles, (2, 0, 1))
    a = a.reshape(3, 2048, 8, 4, 128).transpose(0, 1, 3, 2, 4).reshape(_F)
    mi = null_mask.astype(jnp.int32)
    m = jnp.transpose(mi, (2, 0, 1))
    m = m.reshape(2, 2048, 8, 4, 128).transpose(0, 1, 3, 2, 4).reshape(_M)

    q_flat, i_flat = _sc_call(a, m)

    q = q_flat.reshape(3, 2048, 4, 8, 128).transpose(0, 1, 3, 2, 4)
    q = q.reshape(3, _B, _T).transpose(1, 2, 0)
    ii = i_flat.reshape(3, 2048, 4, 8, 128).transpose(0, 1, 3, 2, 4)
    ii = ii.reshape(3, _B, _T).transpose(1, 2, 0)
    return (q, ii)


# speedup vs baseline: 76.1654x; 1.2086x over previous
import functools
import math

import jax
import jax.numpy as jnp
import numpy as np
from jax import lax
from jax.experimental import pallas as pl
from jax.experimental.pallas import tpu as pltpu
from jax.experimental.pallas import tpu_sc as plsc

_NB = (24, 12, 16)
_PI = math.pi

_B, _T, _C = 16384, 512, 3
_PLANE = _B * _T
_F = _PLANE * _C
_M = _PLANE * 2

_NCORES, _NSUB = 2, 16
_NWORK = _NCORES * _NSUB
_SLAB = _PLANE // _NWORK

_CHUNK = 8192
_NCH = _SLAB // _CHUNK
_UNROLL = 8
_NVEC = _CHUNK // (16 * _UNROLL)
_NBUF = 2


def _sc_body(ang_hbm, msk_hbm, q_hbm, i_hbm,
             ang_v, msk_v, q_v, i_v,
             ain0, ain1, min0, min1, qout0, qout1, iout0, iout1):
    ain, min_ = (ain0, ain1), (min0, min1)
    qout, iout = (qout0, qout1), (iout0, iout1)
    wid = lax.axis_index("s") * _NCORES + lax.axis_index("c")
    sbase = wid * _SLAB

    for ci in range(3):
        n = _NB[ci]
        inv = float(np.float32(n / (2 * _PI)))
        halfn = float(np.float32(n * 0.5))
        width = float(np.float32(2 * _PI / n))
        pbase = ci * _PLANE + sbase
        has_mask = ci < 2

        def start_in(k, b, pbase=pbase, has_mask=has_mask):
            off = pbase + k * _CHUNK
            pltpu.async_copy(ang_hbm.at[pl.ds(off, _CHUNK)], ang_v.at[b], ain[b])
            if has_mask:
                pltpu.async_copy(msk_hbm.at[pl.ds(off, _CHUNK)], msk_v.at[b],
                                 min_[b])

        def wait_in(k, b, pbase=pbase, has_mask=has_mask):
            off = pbase + k * _CHUNK
            pltpu.make_async_copy(ang_hbm.at[pl.ds(off, _CHUNK)], ang_v.at[b],
                                  ain[b]).wait()
            if has_mask:
                pltpu.make_async_copy(msk_hbm.at[pl.ds(off, _CHUNK)],
                                      msk_v.at[b], min_[b]).wait()

        def start_out(k, b, pbase=pbase):
            off = pbase + k * _CHUNK
            pltpu.async_copy(q_v.at[b], q_hbm.at[pl.ds(off, _CHUNK)], qout[b])
            pltpu.async_copy(i_v.at[b], i_hbm.at[pl.ds(off, _CHUNK)], iout[b])

        def wait_out(k, b, pbase=pbase):
            off = pbase + k * _CHUNK
            pltpu.make_async_copy(q_v.at[b], q_hbm.at[pl.ds(off, _CHUNK)],
                                  qout[b]).wait()
            pltpu.make_async_copy(i_v.at[b], i_hbm.at[pl.ds(off, _CHUNK)],
                                  iout[b]).wait()

        def compute(b, ci=ci, n=n, inv=inv, halfn=halfn, width=width):
            def vec(v, c2):
                for u in range(_UNROLL):
                    o = v * (16 * _UNROLL) + u * 16
                    a = ang_v[b, pl.ds(o, 16)]
                    t = a * inv + halfn
                    i = jnp.minimum(t.astype(jnp.int32), n - 1)
                    q = (i.astype(jnp.float32) + 0.5) * width - _PI
                    if ci < 2:
                        m = msk_v[b, pl.ds(o, 16)] != 0
                        q = jnp.where(m, 0.0, q)
                        i = jnp.where(m, n, i)
                    q_v[b, pl.ds(o, 16)] = q
                    i_v[b, pl.ds(o, 16)] = i
                return c2

            lax.fori_loop(0, _NVEC, vec, 0)

        start_in(0, 0)

        def pair(g, carry):
            for b in range(_NBUF):
                k = g + b
                nxt = k + 1

                @pl.when(nxt < _NCH)
                def _():
                    start_in(nxt, 1 - b)

                wait_in(k, b)

                @pl.when(k >= _NBUF)
                def _():
                    wait_out(k - _NBUF, b)

                compute(b)
                start_out(k, b)
            return carry

        lax.fori_loop(0, _NCH // _NBUF, lambda g, c: pair(g * _NBUF, c), 0)
        wait_out(_NCH - 2, 0)
        wait_out(_NCH - 1, 1)


_mesh = plsc.VectorSubcoreMesh(core_axis_name="c", subcore_axis_name="s",
                               num_cores=_NCORES, num_subcores=_NSUB)

_sc_call = functools.partial(
    pl.kernel,
    out_type=(jax.ShapeDtypeStruct((_F,), jnp.float32),
              jax.ShapeDtypeStruct((_F,), jnp.int32)),
    mesh=_mesh,
    scratch_types=[
        pltpu.VMEM((_NBUF, _CHUNK), jnp.float32),
        pltpu.VMEM((_NBUF, _CHUNK), jnp.int32),
        pltpu.VMEM((_NBUF, _CHUNK), jnp.float32),
        pltpu.VMEM((_NBUF, _CHUNK), jnp.int32),
    ] + [pltpu.SemaphoreType.DMA] * 8,
)(_sc_body)


def kernel(angles, null_mask):
    a = jnp.transpose(angles, (2, 0, 1))
    a = a.reshape(3, 2048, 8, 4, 128).transpose(0, 1, 3, 2, 4).reshape(_F)
    mi = null_mask.astype(jnp.int32)
    m = jnp.transpose(mi, (2, 0, 1))
    m = m.reshape(2, 2048, 8, 4, 128).transpose(0, 1, 3, 2, 4).reshape(_M)

    q_flat, i_flat = _sc_call(a, m)

    q = q_flat.reshape(3, 2048, 4, 8, 128).transpose(0, 1, 3, 2, 4)
    q = q.reshape(3, _B, _T).transpose(1, 2, 0)
    ii = i_flat.reshape(3, 2048, 4, 8, 128).transpose(0, 1, 3, 2, 4)
    ii = ii.reshape(3, _B, _T).transpose(1, 2, 0)
    return (q, ii)
